# baseline (speedup 1.0000x reference)
import dataclasses
import functools

import jax
import jax.numpy as jnp
from jax import lax
from jax.experimental import pallas as pl
from jax.experimental.pallas import tpu as pltpu
from jax.experimental.pallas import tpu_sc as plsc

_NC = 2
_NS = 16
_NW = _NC * _NS
_L = 16

_LN2 = 0.6931471805599453
_SQRT2 = 1.4142135623730951


def _ln16(x):
    bits = lax.bitcast_convert_type(x, jnp.int32)
    e = (bits >> 23) - 127
    m = lax.bitcast_convert_type(
        (bits & jnp.int32(0x007FFFFF)) | jnp.int32(0x3F800000), jnp.float32)
    big = m > _SQRT2
    m = jnp.where(big, m * 0.5, m)
    e = jnp.where(big, e + 1, e)
    t = m - 1.0
    z = t * t
    p = jnp.float32(7.0376836292e-2)
    p = p * t + jnp.float32(-1.1514610310e-1)
    p = p * t + jnp.float32(1.1676998740e-1)
    p = p * t + jnp.float32(-1.2420140846e-1)
    p = p * t + jnp.float32(1.4249322787e-1)
    p = p * t + jnp.float32(-1.6668057665e-1)
    p = p * t + jnp.float32(2.0000714765e-1)
    p = p * t + jnp.float32(-2.4999993993e-1)
    p = p * t + jnp.float32(3.3333331174e-1)
    y = t * z * p - 0.5 * z
    return t + y + e.astype(jnp.float32) * jnp.float32(_LN2)


@functools.partial(jax.jit, static_argnames=("n", "c", "p", "ch"))
def _sc_loss(pcl_flat, labels, w, pc_labels, pc_probs, pc_count, img_w,
             im_labels, *, n, c, p, ch):
    n_slices = ch // _L
    p_per_w = p // _NW
    mesh = plsc.VectorSubcoreMesh(core_axis_name="c", subcore_axis_name="s")
    cp = pltpu.CompilerParams()
    if "needs_layout_passes" in pltpu.CompilerParams.__dataclass_fields__:
        cp = dataclasses.replace(cp, needs_layout_passes=False)

    @functools.partial(
        pl.kernel,
        out_type=jax.ShapeDtypeStruct((_NC, _L), jnp.float32),
        mesh=mesh,
        compiler_params=cp,
        scratch_types=[
            pltpu.VMEM((ch,), jnp.float32),
            pltpu.VMEM((ch,), jnp.int32),
            pltpu.VMEM((ch,), jnp.float32),
            pltpu.VMEM((p_per_w,), jnp.int32),
            pltpu.VMEM((p_per_w,), jnp.float32),
            pltpu.VMEM((p_per_w,), jnp.float32),
            pltpu.VMEM((p_per_w,), jnp.float32),
            pltpu.VMEM((c,), jnp.float32),
            pltpu.VMEM((_L,), jnp.float32),
            pltpu.VMEM((_NS * _L,), jnp.float32),
            pltpu.VMEM_SHARED((_NS * _L,), jnp.float32),
            pltpu.SemaphoreType.DMA,
            pltpu.SemaphoreType.DMA,
        ],
    )
    def k(pcl_ref, lab_ref, w_ref, pclab_ref, pcp_ref, pcc_ref, imw_ref,
          im_ref, out_ref, p_v, lab_v, w_v, pclab_v, pcp_v, pcc_v,
          imw_v, im_v, res_v, all_v, stage, sem_bg, sem_fg):
        cid = lax.axis_index("c")
        sid = lax.axis_index("s")
        wid = cid * _NS + sid
        base = jnp.minimum(wid * ch, n - ch)
        start_off = wid * ch - base
        lane = lax.iota(jnp.int32, _L)

        bg_cp = [
            pltpu.async_copy(pcl_ref.at[pl.ds(base, ch)], p_v, sem_bg),
            pltpu.async_copy(lab_ref.at[pl.ds(base, ch)], lab_v, sem_bg),
            pltpu.async_copy(w_ref.at[pl.ds(base, ch)], w_v, sem_bg),
        ]
        pb = wid * p_per_w
        fg_cp = [
            pltpu.async_copy(pclab_ref.at[pl.ds(pb, p_per_w)], pclab_v, sem_fg),
            pltpu.async_copy(pcp_ref.at[pl.ds(pb, p_per_w)], pcp_v, sem_fg),
            pltpu.async_copy(pcc_ref.at[pl.ds(pb, p_per_w)], pcc_v, sem_fg),
            pltpu.async_copy(imw_ref.at[pl.ds(pb, p_per_w)], imw_v, sem_fg),
            pltpu.async_copy(im_ref, im_v, sem_fg),
        ]

        for h in fg_cp:
            h.wait()
        pclab = pclab_v[...]
        im_at = plsc.load_gather(im_v, [jnp.clip(pclab, 0, c - 1)])
        fg_mask = (im_at != 0.0) & (pclab > 0) & (pcc_v[...] > 0.0)
        fg = jnp.where(fg_mask, imw_v[...] * _ln16(pcp_v[...]), 0.0)

        im0 = plsc.load_gather(im_v, [jnp.zeros((_L,), jnp.int32)])
        bg_act = jnp.where(im0 != 0.0, 1.0, 0.0)

        for h in bg_cp:
            h.wait()

        def bg_body(s, acc):
            sl = pl.ds(s * _L, _L)
            off = s * _L + lane
            m = (off >= start_off) & (lab_v[sl] == 0)
            return acc + jnp.where(m, w_v[sl] * _ln16(p_v[sl]), 0.0)

        bg = lax.fori_loop(0, n_slices, bg_body, jnp.zeros((_L,), jnp.float32))

        res_v[...] = bg * bg_act + fg
        pltpu.sync_copy(res_v, stage.at[pl.ds(sid * _L, _L)])
        plsc.subcore_barrier()

        @pl.when(sid == 0)
        def _():
            pltpu.sync_copy(stage, all_v)

            def red_body(r, acc):
                return acc + all_v[pl.ds(r * _L, _L)]

            tot = lax.fori_loop(0, _NS, red_body,
                                jnp.zeros((_L,), jnp.float32))
            val = jnp.sum(tot) * jnp.float32(-1.0 / n)
            res_v[...] = jnp.full((_L,), val, jnp.float32)
            pltpu.sync_copy(res_v, out_ref.at[cid])

    return k(pcl_flat, labels, w, pc_labels, pc_probs, pc_count, img_w,
             im_labels)


def kernel(pcl_prob, labels, cls_loss_weights, gt_assignment, pc_labels,
           pc_probs, pc_count, img_cls_loss_weights, im_labels_real):
    n, c = pcl_prob.shape
    p = pc_labels.shape[0]
    ch = -(-n // (_NW * _L)) * _L
    out = _sc_loss(pcl_prob.T.reshape(-1), labels, cls_loss_weights,
                   pc_labels, pc_probs, pc_count, img_cls_loss_weights,
                   im_labels_real, n=n, c=c, p=p, ch=ch)
    return out[0, 0] + out[1, 0]


# speedup vs baseline: 1.2203x; 1.2203x over previous
import dataclasses
import functools

import jax
import jax.numpy as jnp
from jax import lax
from jax.experimental import pallas as pl
from jax.experimental.pallas import tpu as pltpu
from jax.experimental.pallas import tpu_sc as plsc

_NC = 2
_NS = 16
_NW = _NC * _NS
_L = 16

_LN2 = 0.6931471805599453
_SQRT2 = 1.4142135623730951


def _ln16(x):
    bits = lax.bitcast_convert_type(x, jnp.int32)
    e = (bits >> 23) - 127
    m = lax.bitcast_convert_type(
        (bits & jnp.int32(0x007FFFFF)) | jnp.int32(0x3F800000), jnp.float32)
    big = m > _SQRT2
    m = jnp.where(big, m * 0.5, m)
    e = jnp.where(big, e + 1, e)
    t = m - 1.0
    z = t * t
    p = jnp.float32(7.0376836292e-2)
    p = p * t + jnp.float32(-1.1514610310e-1)
    p = p * t + jnp.float32(1.1676998740e-1)
    p = p * t + jnp.float32(-1.2420140846e-1)
    p = p * t + jnp.float32(1.4249322787e-1)
    p = p * t + jnp.float32(-1.6668057665e-1)
    p = p * t + jnp.float32(2.0000714765e-1)
    p = p * t + jnp.float32(-2.4999993993e-1)
    p = p * t + jnp.float32(3.3333331174e-1)
    y = t * z * p - 0.5 * z
    return t + y + e.astype(jnp.float32) * jnp.float32(_LN2)


@functools.partial(jax.jit, static_argnames=("n", "c", "p", "ch"))
def _sc_loss(pclT, tail, labels, w, pc_labels, pc_probs, pc_count, img_w,
             im_labels, *, n, c, p, ch):
    n128 = (n // 128) * 128
    base_l = (_NW - 1) * ch
    lf = n128 - base_l
    lt = n - n128
    lv = n - base_l
    n_slices = ch // _L
    p_per_w = p // _NW
    mesh = plsc.VectorSubcoreMesh(core_axis_name="c", subcore_axis_name="s")
    cp = pltpu.CompilerParams()
    if "needs_layout_passes" in pltpu.CompilerParams.__dataclass_fields__:
        cp = dataclasses.replace(cp, needs_layout_passes=False)

    @functools.partial(
        pl.kernel,
        out_type=jax.ShapeDtypeStruct((_NC, _L), jnp.float32),
        mesh=mesh,
        compiler_params=cp,
        scratch_types=[
            pltpu.VMEM((8, ch), jnp.float32),
            pltpu.VMEM((ch,), jnp.int32),
            pltpu.VMEM((ch,), jnp.float32),
            pltpu.VMEM((p_per_w,), jnp.int32),
            pltpu.VMEM((p_per_w,), jnp.float32),
            pltpu.VMEM((p_per_w,), jnp.float32),
            pltpu.VMEM((p_per_w,), jnp.float32),
            pltpu.VMEM((c,), jnp.float32),
            pltpu.VMEM((_L,), jnp.float32),
            pltpu.VMEM((_NS * _L,), jnp.float32),
            pltpu.VMEM_SHARED((_NS * _L,), jnp.float32),
            pltpu.SemaphoreType.DMA,
            pltpu.SemaphoreType.DMA,
        ],
    )
    def k(pcl_ref, tail_ref, lab_ref, w_ref, pclab_ref, pcp_ref, pcc_ref,
          imw_ref, im_ref, out_ref, p_v, lab_v, w_v, pclab_v, pcp_v, pcc_v,
          imw_v, im_v, res_v, all_v, stage, sem_bg, sem_fg):
        cid = lax.axis_index("c")
        sid = lax.axis_index("s")
        wid = cid * _NS + sid
        base = wid * ch
        valid = jnp.minimum(n - base, ch)
        lane = lax.iota(jnp.int32, _L)

        pb = wid * p_per_w
        fg_cp = [
            pltpu.async_copy(pclab_ref.at[pl.ds(pb, p_per_w)], pclab_v, sem_fg),
            pltpu.async_copy(pcp_ref.at[pl.ds(pb, p_per_w)], pcp_v, sem_fg),
            pltpu.async_copy(pcc_ref.at[pl.ds(pb, p_per_w)], pcc_v, sem_fg),
            pltpu.async_copy(imw_ref.at[pl.ds(pb, p_per_w)], imw_v, sem_fg),
            pltpu.async_copy(im_ref, im_v, sem_fg),
        ]

        @pl.when(wid != _NW - 1)
        def _():
            h = [
                pltpu.async_copy(
                    pcl_ref.at[pl.ds(0, 8), pl.ds(base, ch)], p_v, sem_bg),
                pltpu.async_copy(lab_ref.at[pl.ds(base, ch)], lab_v, sem_bg),
                pltpu.async_copy(w_ref.at[pl.ds(base, ch)], w_v, sem_bg),
            ]
            for x in h:
                x.wait()

        @pl.when(wid == _NW - 1)
        def _():
            h = [
                pltpu.async_copy(
                    pcl_ref.at[pl.ds(0, 8), pl.ds(base_l, lf)],
                    p_v.at[pl.ds(0, 8), pl.ds(0, lf)], sem_bg),
                pltpu.async_copy(tail_ref, p_v.at[0, pl.ds(lf, lt)], sem_bg),
                pltpu.async_copy(lab_ref.at[pl.ds(base_l, lv)],
                                 lab_v.at[pl.ds(0, lv)], sem_bg),
                pltpu.async_copy(w_ref.at[pl.ds(base_l, lv)],
                                 w_v.at[pl.ds(0, lv)], sem_bg),
            ]
            for x in h:
                x.wait()

        for h in fg_cp:
            h.wait()
        pclab = pclab_v[...]
        im_at = plsc.load_gather(im_v, [jnp.clip(pclab, 0, c - 1)])
        fg_mask = (im_at != 0.0) & (pclab > 0) & (pcc_v[...] > 0.0)
        fg = jnp.where(fg_mask, imw_v[...] * _ln16(pcp_v[...]), 0.0)

        im0 = plsc.load_gather(im_v, [jnp.zeros((_L,), jnp.int32)])
        bg_act = jnp.where(im0 != 0.0, 1.0, 0.0)

        def bg_body(s, acc):
            sl = pl.ds(s * _L, _L)
            off = s * _L + lane
            m = (off < valid) & (lab_v[sl] == 0)
            return acc + jnp.where(m, w_v[sl] * _ln16(p_v[0, sl]), 0.0)

        bg = lax.fori_loop(0, n_slices, bg_body, jnp.zeros((_L,), jnp.float32))

        res_v[...] = bg * bg_act + fg
        pltpu.sync_copy(res_v, stage.at[pl.ds(sid * _L, _L)])
        plsc.subcore_barrier()

        @pl.when(sid == 0)
        def _():
            pltpu.sync_copy(stage, all_v)

            def red_body(r, acc):
                return acc + all_v[pl.ds(r * _L, _L)]

            tot = lax.fori_loop(0, _NS, red_body,
                                jnp.zeros((_L,), jnp.float32))
            val = jnp.sum(tot) * jnp.float32(-1.0 / n)
            res_v[...] = jnp.full((_L,), val, jnp.float32)
            pltpu.sync_copy(res_v, out_ref.at[cid])

    return k(pclT, tail, labels, w, pc_labels, pc_probs, pc_count, img_w,
             im_labels)


def kernel(pcl_prob, labels, cls_loss_weights, gt_assignment, pc_labels,
           pc_probs, pc_count, img_cls_loss_weights, im_labels_real):
    n, c = pcl_prob.shape
    p = pc_labels.shape[0]
    ch = -(-n // (_NW * 128)) * 128
    pclT = pcl_prob.T
    n128 = (n // 128) * 128
    tail = pclT[0, n128:]
    out = _sc_loss(pclT, tail, labels, cls_loss_weights,
                   pc_labels, pc_probs, pc_count, img_cls_loss_weights,
                   im_labels_real, n=n, c=c, p=p, ch=ch)
    return out[0, 0] + out[1, 0]


# speedup vs baseline: 14.5915x; 11.9574x over previous
import functools

import jax
import jax.numpy as jnp
from jax import lax
from jax.experimental import pallas as pl


def _body(pclT_ref, lab_ref, w_ref, pclab_ref, pcp_ref, pcc_ref, imw_ref,
          im_ref, out_ref, *, n, c, p):
    z = jnp.log(pclT_ref[0:1, :]).reshape(n)
    wm = jnp.where(lab_ref[...] == 0, w_ref[...], 0.0)
    bg = jnp.sum(wm * z, keepdims=True).reshape(1, 1)

    im_r = im_ref[...].reshape(1, c)
    gate = (im_r[:, 0:1] != 0.0).astype(jnp.float32)

    pclab = pclab_ref[...].reshape(1, p)
    imnz = (im_r != 0.0).astype(jnp.float32)
    onehot = (lax.broadcasted_iota(jnp.int32, (c, p), 0)
              == pclab).astype(jnp.float32)
    im_at_nz = lax.dot_general(
        imnz, onehot,
        dimension_numbers=(((1,), (0,)), ((), ())),
        preferred_element_type=jnp.float32)
    fg_mask = ((im_at_nz > 0.5) & (pclab > 0)
               & (pcc_ref[...].reshape(1, p) > 0.0))
    fg = jnp.sum(
        jnp.where(fg_mask,
                  imw_ref[...].reshape(1, p) * jnp.log(pcp_ref[...].reshape(1, p)),
                  0.0),
        keepdims=True)

    out_ref[...] = (gate * bg + fg) * jnp.float32(-1.0 / n)


@functools.partial(jax.jit, static_argnames=("n", "c", "p"))
def _loss(pclT, labels, w, pc_labels, pc_probs, pc_count, img_w,
          im_labels, *, n, c, p):
    full1 = lambda i: (0,)
    out = pl.pallas_call(
        functools.partial(_body, n=n, c=c, p=p),
        grid=(1,),
        in_specs=[
            pl.BlockSpec((8, n), lambda i: (0, 0)),
            pl.BlockSpec((n,), full1),
            pl.BlockSpec((n,), full1),
            pl.BlockSpec((p,), full1),
            pl.BlockSpec((p,), full1),
            pl.BlockSpec((p,), full1),
            pl.BlockSpec((p,), full1),
            pl.BlockSpec((c,), full1),
        ],
        out_specs=pl.BlockSpec((1, 1), lambda i: (0, 0)),
        out_shape=jax.ShapeDtypeStruct((1, 1), jnp.float32),
    )(pclT, labels, w, pc_labels, pc_probs, pc_count, img_w, im_labels)
    return out[0, 0]


def kernel(pcl_prob, labels, cls_loss_weights, gt_assignment, pc_labels,
           pc_probs, pc_count, img_cls_loss_weights, im_labels_real):
    n, c = pcl_prob.shape
    p = pc_labels.shape[0]
    return _loss(pcl_prob.T, labels, cls_loss_weights, pc_labels, pc_probs,
                 pc_count, img_cls_loss_weights, im_labels_real,
                 n=n, c=c, p=p)
